# baseline (speedup 1.0000x reference)
import functools

import jax
import jax.numpy as jnp
from jax import lax
from jax.experimental import pallas as pl
from jax.experimental.pallas import tpu as pltpu
from jax.experimental.pallas import tpu_sc as plsc

EMB = 32
PER_DMA = 2048
K = 1
CHUNK = K * PER_DMA
NW = 32


@functools.partial(jax.jit, static_argnums=(2,))
def _lookup(idx2d, table, total):
    per_w = total // NW
    groups = per_w // CHUNK
    mesh = plsc.VectorSubcoreMesh(core_axis_name="c", subcore_axis_name="s")

    @functools.partial(
        pl.kernel,
        out_type=jax.ShapeDtypeStruct((total, EMB), jnp.float32),
        mesh=mesh,
        scratch_types=[
            pltpu.VMEM((K, PER_DMA), jnp.int32),
            pltpu.VMEM((CHUNK, EMB), jnp.float32),
            pltpu.SemaphoreType.DMA,
        ],
        compiler_params=pltpu.CompilerParams(use_tc_tiling_on_sc=False),
    )
    def body(table_hbm, idx_hbm, out_hbm, idx_v, rows_v, sem):
        wid = lax.axis_index("s") * 2 + lax.axis_index("c")
        row_base = wid * (per_w // PER_DMA)
        out_base = wid * per_w

        def step(g, carry):
            pltpu.sync_copy(idx_hbm.at[pl.ds(row_base + g * K, K)], idx_v)
            copies = [
                pltpu.async_copy(
                    table_hbm.at[idx_v.at[j]],
                    rows_v.at[pl.ds(j * PER_DMA, PER_DMA)],
                    sem,
                )
                for j in range(K)
            ]
            for c in copies:
                c.wait()
            pltpu.sync_copy(
                rows_v, out_hbm.at[pl.ds(out_base + g * CHUNK, CHUNK)]
            )
            return carry

        lax.fori_loop(0, groups, step, 0)

    return body(table, idx2d)


def kernel(time_indices, table):
    b, t = time_indices.shape
    total = b * t
    idx2d = time_indices.reshape(total // PER_DMA, PER_DMA)
    out = _lookup(idx2d, table, total)
    return out.reshape(b, t, EMB)


# speedup vs baseline: 1.0289x; 1.0289x over previous
import functools

import jax
import jax.numpy as jnp
from jax import lax
from jax.experimental import pallas as pl
from jax.experimental.pallas import tpu as pltpu
from jax.experimental.pallas import tpu_sc as plsc

EMB = 32
CHUNK = 1600
NW = 32


@functools.partial(jax.jit, static_argnums=(2,))
def _lookup(idx_flat, table, total):
    per_w = total // NW
    groups = per_w // CHUNK
    assert groups >= 6 and groups % 2 == 0
    mesh = plsc.VectorSubcoreMesh(core_axis_name="c", subcore_axis_name="s")

    @functools.partial(
        pl.kernel,
        out_type=jax.ShapeDtypeStruct((total, EMB), jnp.float32),
        mesh=mesh,
        scratch_types=[
            pltpu.VMEM((2, CHUNK), jnp.int32),
            pltpu.VMEM((2, CHUNK, EMB), jnp.float32),
            pltpu.SemaphoreType.DMA,
            pltpu.SemaphoreType.DMA,
            pltpu.SemaphoreType.DMA,
            pltpu.SemaphoreType.DMA,
            pltpu.SemaphoreType.DMA,
            pltpu.SemaphoreType.DMA,
        ],
        compiler_params=pltpu.CompilerParams(use_tc_tiling_on_sc=False),
    )
    def body(table_hbm, idx_hbm, out_hbm, idx_v, rows_v,
             si0, si1, sg0, sg1, so0, so1):
        wid = lax.axis_index("s") * 2 + lax.axis_index("c")
        base = wid * per_w
        i0, i1 = idx_v.at[0], idx_v.at[1]
        r0, r1 = rows_v.at[0], rows_v.at[1]

        def start_idx(g, buf, sem):
            return pltpu.async_copy(
                idx_hbm.at[pl.ds(base + g * CHUNK, CHUNK)], buf, sem)

        def start_gather(buf, rows, sem):
            return pltpu.async_copy(table_hbm.at[buf], rows, sem)

        def start_store(rows, g, sem):
            return pltpu.async_copy(
                rows, out_hbm.at[pl.ds(base + g * CHUNK, CHUNK)], sem)

        def wait_idx(buf, sem):
            pltpu.make_async_copy(
                idx_hbm.at[pl.ds(base, CHUNK)], buf, sem).wait()

        def wait_gather(rows, sem):
            pltpu.make_async_copy(
                out_hbm.at[pl.ds(base, CHUNK)], rows, sem).wait()

        def wait_store(rows, sem):
            pltpu.make_async_copy(
                rows, out_hbm.at[pl.ds(base, CHUNK)], sem).wait()

        d_ia = start_idx(0, i0, si0)
        d_ib = start_idx(1, i1, si1)
        d_ia.wait()
        d_g0 = start_gather(i0, r0, sg0)

        d_g0.wait()
        st0 = start_store(r0, 0, so0)
        d_i2 = start_idx(2, i0, si0)
        d_ib.wait()
        d_g1 = start_gather(i1, r1, sg1)

        d_g1.wait()
        st1 = start_store(r1, 1, so1)
        d_i3 = start_idx(3, i1, si1)
        d_i2.wait()
        st0.wait()
        start_gather(i0, r0, sg0)

        def step(it, carry):
            g = 2 + 2 * it
            wait_gather(r0, sg0)
            st_a = start_store(r0, g, so0)
            d_in = start_idx(g + 2, i0, si0)
            wait_idx(i1, si1)
            wait_store(r1, so1)
            d_g = start_gather(i1, r1, sg1)
            d_g.wait()
            start_store(r1, g + 1, so1)
            start_idx(g + 3, i1, si1)
            d_in.wait()
            st_a.wait()
            start_gather(i0, r0, sg0)
            return carry

        lax.fori_loop(0, (groups - 4) // 2, step, 0)

        wait_gather(r0, sg0)
        st_a = start_store(r0, groups - 2, so0)
        wait_idx(i1, si1)
        wait_store(r1, so1)
        d_g = start_gather(i1, r1, sg1)

        d_g.wait()
        st_b = start_store(r1, groups - 1, so1)
        st_a.wait()
        st_b.wait()

    return body(table, idx_flat)


def kernel(time_indices, table):
    b, t = time_indices.shape
    total = b * t
    out = _lookup(time_indices.reshape(total), table, total)
    return out.reshape(b, t, EMB)
